# baseline (speedup 1.0000x reference)
import functools

import jax
import jax.numpy as jnp
from jax import lax
from jax.experimental import pallas as pl
from jax.experimental.pallas import tpu as pltpu
from jax.experimental.pallas import tpu_sc as plsc

N_USERS = 25000
N_NODES = 50000
N_PAD = 50048
E = 800000
E_PAD = 819200
D = 64
DH = 32
PK = 128
B = 4096
MLP_H = 32

EPT = E_PAD // 16
C_AGG = 256
NC_AGG = EPT // C_AGG
NS_AGG = 3
EPW = E_PAD // 32
C_DEG = 512
NC_DEG = EPW // C_DEG
ROWS_PER_TILE = N_PAD // 16

_MESH = plsc.VectorSubcoreMesh(core_axis_name="c", subcore_axis_name="s")
_SC_PARAMS = pltpu.CompilerParams(use_tc_tiling_on_sc=False)


def _zero_fill(ref, nrows, ncols):
    def body(r, _):
        for col in range(ncols // 16):
            ref[r, pl.ds(col * 16, 16)] = jnp.zeros((16,), jnp.float32)
        return 0
    lax.fori_loop(0, nrows, body, 0)


def _zero_acc(acc, chunk, chunk_rows, base):
    full, rem = ROWS_PER_TILE // chunk_rows, ROWS_PER_TILE % chunk_rows
    for t in range(full):
        pltpu.sync_copy(chunk.at[pl.ds(0, chunk_rows)],
                        acc.at[pl.ds(base + t * chunk_rows, chunk_rows)])
    if rem:
        pltpu.sync_copy(chunk.at[pl.ds(0, rem)],
                        acc.at[pl.ds(base + full * chunk_rows, rem)])


@functools.partial(
    pl.kernel,
    out_type=jax.ShapeDtypeStruct((N_PAD, 8, 16), jnp.float32),
    mesh=_MESH,
    compiler_params=_SC_PARAMS,
    scratch_types=[
        pltpu.VMEM_SHARED((N_PAD, 16), jnp.float32),
        pltpu.VMEM((C_DEG, 16), jnp.float32),
        pltpu.VMEM((2, C_DEG), jnp.int32),
        pltpu.SemaphoreType.DMA((2,)),
        pltpu.SemaphoreType.DMA((2,)),
    ],
)
def _sc_degree(dst1d, out, deg_acc, ones_v, idx_v, ssem, isem):
    c = lax.axis_index("c")
    s = lax.axis_index("s")
    w = c * 16 + s
    base = s * ROWS_PER_TILE
    ebase = w * EPW

    _zero_fill(ones_v, C_DEG, 16)
    _zero_acc(deg_acc, ones_v, C_DEG, base)

    one = jnp.ones((16,), jnp.float32)

    def fill(r, _):
        ones_v[r, pl.ds(0, 16)] = one
        return 0
    lax.fori_loop(0, C_DEG, fill, 0)

    pltpu.sync_copy(dst1d.at[pl.ds(ebase, C_DEG)], idx_v.at[0])
    pltpu.make_async_copy(dst1d.at[pl.ds(ebase + C_DEG, C_DEG)],
                          idx_v.at[1], isem.at[1]).start()
    plsc.subcore_barrier()

    def body(i, _):
        slot = lax.rem(i, 2)
        sdst = deg_acc.at[idx_v.at[slot]]
        pltpu.make_async_copy(ones_v, sdst, ssem.at[slot]).start(add=True)
        pltpu.make_async_copy(ones_v, sdst, ssem.at[slot]).wait()

        @pl.when(i + 2 < NC_DEG)
        def _():
            pltpu.make_async_copy(
                dst1d.at[pl.ds(ebase + (i + 2) * C_DEG, C_DEG)],
                idx_v.at[slot], isem.at[slot]).start()

        @pl.when(i + 1 < NC_DEG)
        def _():
            pltpu.make_async_copy(
                dst1d.at[pl.ds(ebase + (i + 1) * C_DEG, C_DEG)],
                idx_v.at[1 - slot], isem.at[1 - slot]).wait()
        return 0

    lax.fori_loop(0, NC_DEG, body, 0)

    plsc.subcore_barrier()
    pltpu.sync_copy(deg_acc.at[pl.ds(base, ROWS_PER_TILE)],
                    out.at[pl.ds(base, ROWS_PER_TILE), c])


@functools.partial(
    pl.kernel,
    out_type=jax.ShapeDtypeStruct((N_PAD, 4, DH), jnp.float32),
    mesh=_MESH,
    compiler_params=_SC_PARAMS,
    scratch_types=[
        pltpu.VMEM_SHARED((N_PAD, DH), jnp.float32),
        pltpu.VMEM((NS_AGG, C_AGG, DH), jnp.float32),
        pltpu.VMEM((NS_AGG, C_AGG), jnp.int32),
        pltpu.VMEM((NS_AGG, C_AGG), jnp.int32),
        pltpu.SemaphoreType.DMA((NS_AGG,)),
        pltpu.SemaphoreType.DMA((NS_AGG,)),
        pltpu.SemaphoreType.DMA((NS_AGG, 2)),
    ],
)
def _sc_aggregate(src4, dst1d, table4, out,
                  acc, rows, sidx, didx, gsem, ssem, isem):
    c = lax.axis_index("c")
    s = lax.axis_index("s")
    base = s * ROWS_PER_TILE
    ebase = s * EPT

    _zero_fill(rows.at[0], C_AGG, DH)
    _zero_acc(acc, rows.at[0], C_AGG, base)

    src1d = src4.at[c]

    def idx_copy(i, slot):
        off = ebase + i * C_AGG
        return (
            pltpu.make_async_copy(src1d.at[pl.ds(off, C_AGG)],
                                  sidx.at[slot], isem.at[slot, 0]),
            pltpu.make_async_copy(dst1d.at[pl.ds(off, C_AGG)],
                                  didx.at[slot], isem.at[slot, 1]),
        )

    def run():
        def gather(i, slot):
            return pltpu.make_async_copy(
                table4.at[sidx.at[slot]], rows.at[slot], gsem.at[slot])

        def scatter(slot):
            return pltpu.make_async_copy(
                rows.at[slot], acc.at[didx.at[slot]], ssem.at[slot])

        a, b_ = idx_copy(0, 0)
        a.start(); b_.start(); a.wait(); b_.wait()
        for j in (1, 2):
            a, b_ = idx_copy(j, j)
            a.start(); b_.start()
        gather(0, 0).start()
        a, b_ = idx_copy(1, 1)
        a.wait(); b_.wait()
        gather(1, 1).start()

        def body(i, _):
            slot = lax.rem(i, NS_AGG)
            gather(i, slot).wait()
            scatter(slot).start(add=True)

            @pl.when(i + 2 < NC_AGG)
            def _():
                nslot = lax.rem(i + 2, NS_AGG)
                a, b_ = idx_copy(i + 2, nslot)
                a.wait()
                b_.wait()
                gather(i + 2, nslot).start()

            scatter(slot).wait()

            @pl.when(i + 3 < NC_AGG)
            def _():
                a, b_ = idx_copy(i + 3, slot)
                a.start()
                b_.start()
            return 0

        lax.fori_loop(0, NC_AGG, body, 0)

    run()

    plsc.subcore_barrier()
    pltpu.sync_copy(acc.at[pl.ds(base, ROWS_PER_TILE)],
                    out.at[pl.ds(base, ROWS_PER_TILE), c])


@functools.partial(
    pl.kernel,
    out_type=(jax.ShapeDtypeStruct((B, PK), jnp.float32),
              jax.ShapeDtypeStruct((B, PK), jnp.float32)),
    mesh=_MESH,
    compiler_params=_SC_PARAMS,
    scratch_types=[
        pltpu.VMEM((256,), jnp.int32),
        pltpu.VMEM((256, PK), jnp.float32),
        pltpu.SemaphoreType.DMA,
    ],
)
def _sc_decode_gather(x3, uids, iids, outu, outi, idx_v, rows_v, sem):
    c = lax.axis_index("c")
    s = lax.axis_index("s")
    base = s * 256

    @pl.when(c == 0)
    def _():
        pltpu.sync_copy(uids.at[pl.ds(base, 256)], idx_v)

    @pl.when(c == 1)
    def _():
        pltpu.sync_copy(iids.at[pl.ds(base, 256)], idx_v)
        off = jnp.full((16,), N_USERS, jnp.int32)

        def addoff(j, _):
            idx_v[pl.ds(j * 16, 16)] = idx_v[pl.ds(j * 16, 16)] + off
            return 0
        lax.fori_loop(0, 16, addoff, 0)

    pltpu.make_async_copy(x3.at[idx_v], rows_v, sem).start()
    pltpu.make_async_copy(x3.at[idx_v], rows_v, sem).wait()

    @pl.when(c == 0)
    def _():
        pltpu.sync_copy(rows_v, outu.at[pl.ds(base, 256)])

    @pl.when(c == 1)
    def _():
        pltpu.sync_copy(rows_v, outi.at[pl.ds(base, 256)])


_RB = 3128
_GRID = N_PAD // _RB
_PREC = lax.Precision.HIGHEST


def _tc_transform0_body(x_ref, wlT_ref, wrT_ref, bl_ref, t_ref):
    x = x_ref[...]
    y = jnp.dot(x, wlT_ref[...], precision=_PREC)
    z = jnp.dot(x, wrT_ref[...], precision=_PREC) + bl_ref[...]
    t_ref[...] = jnp.concatenate([y, z], axis=1)


def _node_block(acc_ref, deg_ref, tp_ref):
    deg = deg_ref[:, :1] + deg_ref[:, 16:17]
    inv = 1.0 / jnp.maximum(deg, 1.0)
    agg = acc_ref[:, :D] * inv
    return jnp.maximum(agg + tp_ref[:, D:], 0.0)


def _tc_transform_body(acc_ref, deg_ref, tp_ref,
                       wlT_ref, wrT_ref, bl_ref, t_ref):
    x = _node_block(acc_ref, deg_ref, tp_ref)
    y = jnp.dot(x, wlT_ref[...], precision=_PREC)
    z = jnp.dot(x, wrT_ref[...], precision=_PREC) + bl_ref[...]
    t_ref[...] = jnp.concatenate([y, z], axis=1)


def _tc_combine_body(acc_ref, deg_ref, tp_ref, x_ref):
    x = _node_block(acc_ref, deg_ref, tp_ref)
    x_ref[...] = jnp.concatenate([x, x], axis=1)


def _tc_mlp_body(gu_ref, gi_ref, w1uT_ref, w1iT_ref, b1_ref, w2T_ref, b2_ref,
                 out_ref):
    h = (jnp.dot(gu_ref[:, :D], w1uT_ref[...], precision=_PREC)
         + jnp.dot(gi_ref[:, :D], w1iT_ref[...], precision=_PREC)
         + b1_ref[...])
    h = jnp.maximum(h, 0.0)
    r = jnp.dot(h, w2T_ref[...], precision=_PREC) + b2_ref[...]
    out_ref[...] = jnp.clip(r, 1.0, 5.0)


def _row_spec(cols, col0=0):
    cb = col0 // max(cols, 1)
    return pl.BlockSpec((_RB, cols), lambda i, _cb=cb: (i, _cb))


def _full_spec(r, cols):
    return pl.BlockSpec((r, cols), lambda i: (0, 0))


def _tc_transform0(x, wlT, wrT, bl):
    return pl.pallas_call(
        _tc_transform0_body,
        grid=(_GRID,),
        in_specs=[_row_spec(D), _full_spec(D, D), _full_spec(D, D),
                  _full_spec(1, D)],
        out_specs=_row_spec(PK),
        out_shape=jax.ShapeDtypeStruct((N_PAD, PK), jnp.float32),
    )(x, wlT, wrT, bl)


def _tc_transform(acc2, deg2, tp, wlT, wrT, bl):
    return pl.pallas_call(
        _tc_transform_body,
        grid=(_GRID,),
        in_specs=[_row_spec(PK), _row_spec(PK), _row_spec(PK),
                  _full_spec(D, D), _full_spec(D, D), _full_spec(1, D)],
        out_specs=_row_spec(PK),
        out_shape=jax.ShapeDtypeStruct((N_PAD, PK), jnp.float32),
    )(acc2, deg2, tp, wlT, wrT, bl)


def _tc_combine(acc2, deg2, tp):
    return pl.pallas_call(
        _tc_combine_body,
        grid=(_GRID,),
        in_specs=[_row_spec(PK), _row_spec(PK), _row_spec(PK)],
        out_specs=_row_spec(PK),
        out_shape=jax.ShapeDtypeStruct((N_PAD, PK), jnp.float32),
    )(acc2, deg2, tp)


def _tc_mlp(gu, gi, w1uT, w1iT, b1, w2T, b2):
    return pl.pallas_call(
        _tc_mlp_body,
        grid=(1,),
        in_specs=[_full_spec(B, PK), _full_spec(B, PK), _full_spec(D, MLP_H),
                  _full_spec(D, MLP_H), _full_spec(1, MLP_H),
                  _full_spec(MLP_H, 1), _full_spec(1, 1)],
        out_specs=_full_spec(B, 1),
        out_shape=jax.ShapeDtypeStruct((B, 1), jnp.float32),
    )(gu, gi, w1uT, w1iT, b1, w2T, b2)


def kernel(edge_index, user_ids, item_ids, user_emb, item_emb,
           Wl0, bl0, Wr0, Wl1, bl1, Wr1, Wl2, bl2, Wr2,
           W1, b1, W2, b2):
    x0 = jnp.concatenate([user_emb, item_emb], axis=0)
    x0 = jnp.pad(x0, ((0, N_PAD - N_NODES), (0, 0)))
    pad_i = jnp.arange(E_PAD - E, dtype=jnp.int32)
    src_pad = (pad_i * 37) % N_NODES
    dst_pad = N_NODES + pad_i % (N_PAD - N_NODES)
    src_p = jnp.concatenate([edge_index[0], src_pad])
    dst_p = jnp.concatenate([edge_index[1], dst_pad])
    src4 = jnp.stack([src_p * 4, src_p * 4 + 1])

    deg2 = _sc_degree(dst_p).reshape(N_PAD, PK)

    layers = ((Wl0, bl0, Wr0), (Wl1, bl1, Wr1), (Wl2, bl2, Wr2))
    acc2 = t = None
    for k, (Wl, bl, Wr) in enumerate(layers):
        wlT, wrT, bl2d = Wl.T, Wr.T, bl.reshape(1, D)
        if k == 0:
            t = _tc_transform0(x0, wlT, wrT, bl2d)
        else:
            t = _tc_transform(acc2, deg2, t, wlT, wrT, bl2d)
        t4 = lax.optimization_barrier(t.reshape(4 * N_PAD, DH))
        acc2 = _sc_aggregate(src4, dst_p, t4).reshape(N_PAD, PK)

    x3 = _tc_combine(acc2, deg2, t)
    gu, gi = _sc_decode_gather(x3, user_ids, item_ids)

    w1uT, w1iT = W1[:, :D].T, W1[:, D:].T
    out = _tc_mlp(gu, gi, w1uT, w1iT, b1.reshape(1, MLP_H),
                  W2.T, b2.reshape(1, 1))
    return out.reshape(B)


# speedup vs baseline: 1.7214x; 1.7214x over previous
import functools

import jax
import jax.numpy as jnp
from jax import lax
from jax.experimental import pallas as pl
from jax.experimental.pallas import tpu as pltpu
from jax.experimental.pallas import tpu_sc as plsc

N_USERS = 25000
N_NODES = 50000
N_PAD = 50048
E = 800000
E_PAD = 819200
D = 64
DH = 32
PK = 128
B = 4096
MLP_H = 32

EPT = E_PAD // 16
C_AGG = 256
NC_AGG = EPT // C_AGG
NS_AGG = 3
EPW = E_PAD // 32
C_DEG = 512
NC_DEG = EPW // C_DEG
ROWS_PER_TILE = N_PAD // 16

_MESH = plsc.VectorSubcoreMesh(core_axis_name="c", subcore_axis_name="s")
_SC_PARAMS = pltpu.CompilerParams(use_tc_tiling_on_sc=False)


def _zero_fill(ref, nrows, ncols):
    def body(r, _):
        for col in range(ncols // 16):
            ref[r, pl.ds(col * 16, 16)] = jnp.zeros((16,), jnp.float32)
        return 0
    lax.fori_loop(0, nrows, body, 0)


def _zero_acc(acc, chunk, chunk_rows, base):
    full, rem = ROWS_PER_TILE // chunk_rows, ROWS_PER_TILE % chunk_rows
    for t in range(full):
        pltpu.sync_copy(chunk.at[pl.ds(0, chunk_rows)],
                        acc.at[pl.ds(base + t * chunk_rows, chunk_rows)])
    if rem:
        pltpu.sync_copy(chunk.at[pl.ds(0, rem)],
                        acc.at[pl.ds(base + full * chunk_rows, rem)])


@functools.partial(
    pl.kernel,
    out_type=jax.ShapeDtypeStruct((2, N_PAD, 16), jnp.float32),
    mesh=_MESH,
    compiler_params=_SC_PARAMS,
    scratch_types=[
        pltpu.VMEM_SHARED((N_PAD, 16), jnp.float32),
        pltpu.VMEM((C_DEG, 16), jnp.float32),
        pltpu.VMEM((2, C_DEG), jnp.int32),
        pltpu.SemaphoreType.DMA((2,)),
        pltpu.SemaphoreType.DMA((2,)),
    ],
)
def _sc_degree(dst1d, out, deg_acc, ones_v, idx_v, ssem, isem):
    c = lax.axis_index("c")
    s = lax.axis_index("s")
    w = c * 16 + s
    base = s * ROWS_PER_TILE
    ebase = w * EPW

    _zero_fill(ones_v, C_DEG, 16)
    _zero_acc(deg_acc, ones_v, C_DEG, base)

    one = jnp.ones((16,), jnp.float32)

    def fill(r, _):
        ones_v[r, pl.ds(0, 16)] = one
        return 0
    lax.fori_loop(0, C_DEG, fill, 0)

    pltpu.sync_copy(dst1d.at[pl.ds(ebase, C_DEG)], idx_v.at[0])
    pltpu.make_async_copy(dst1d.at[pl.ds(ebase + C_DEG, C_DEG)],
                          idx_v.at[1], isem.at[1]).start()
    plsc.subcore_barrier()

    def body(i, _):
        slot = lax.rem(i, 2)
        sdst = deg_acc.at[idx_v.at[slot]]
        pltpu.make_async_copy(ones_v, sdst, ssem.at[slot]).start(add=True)
        pltpu.make_async_copy(ones_v, sdst, ssem.at[slot]).wait()

        @pl.when(i + 2 < NC_DEG)
        def _():
            pltpu.make_async_copy(
                dst1d.at[pl.ds(ebase + (i + 2) * C_DEG, C_DEG)],
                idx_v.at[slot], isem.at[slot]).start()

        @pl.when(i + 1 < NC_DEG)
        def _():
            pltpu.make_async_copy(
                dst1d.at[pl.ds(ebase + (i + 1) * C_DEG, C_DEG)],
                idx_v.at[1 - slot], isem.at[1 - slot]).wait()
        return 0

    lax.fori_loop(0, NC_DEG, body, 0)

    plsc.subcore_barrier()
    pltpu.sync_copy(deg_acc.at[pl.ds(base, ROWS_PER_TILE)],
                    out.at[c, pl.ds(base, ROWS_PER_TILE)])


@functools.partial(
    pl.kernel,
    out_type=(jax.ShapeDtypeStruct((N_PAD, DH), jnp.float32),
              jax.ShapeDtypeStruct((N_PAD, DH), jnp.float32)),
    mesh=_MESH,
    compiler_params=_SC_PARAMS,
    scratch_types=[
        pltpu.VMEM_SHARED((N_PAD, DH), jnp.float32),
        pltpu.VMEM((NS_AGG, C_AGG, DH), jnp.float32),
        pltpu.VMEM((NS_AGG, C_AGG), jnp.int32),
        pltpu.VMEM((NS_AGG, C_AGG), jnp.int32),
        pltpu.SemaphoreType.DMA((NS_AGG,)),
        pltpu.SemaphoreType.DMA((NS_AGG,)),
        pltpu.SemaphoreType.DMA((NS_AGG, 2)),
    ],
)
def _sc_aggregate(src4, dst1d, table4, outa, outb,
                  acc, rows, sidx, didx, gsem, ssem, isem):
    c = lax.axis_index("c")
    s = lax.axis_index("s")
    base = s * ROWS_PER_TILE
    ebase = s * EPT

    _zero_fill(rows.at[0], C_AGG, DH)
    _zero_acc(acc, rows.at[0], C_AGG, base)

    src1d = src4.at[c]

    def idx_copy(i, slot):
        off = ebase + i * C_AGG
        return (
            pltpu.make_async_copy(src1d.at[pl.ds(off, C_AGG)],
                                  sidx.at[slot], isem.at[slot, 0]),
            pltpu.make_async_copy(dst1d.at[pl.ds(off, C_AGG)],
                                  didx.at[slot], isem.at[slot, 1]),
        )

    def run():
        def gather(i, slot):
            return pltpu.make_async_copy(
                table4.at[sidx.at[slot]], rows.at[slot], gsem.at[slot])

        def scatter(slot):
            return pltpu.make_async_copy(
                rows.at[slot], acc.at[didx.at[slot]], ssem.at[slot])

        a, b_ = idx_copy(0, 0)
        a.start(); b_.start(); a.wait(); b_.wait()
        for j in (1, 2):
            a, b_ = idx_copy(j, j)
            a.start(); b_.start()
        gather(0, 0).start()
        a, b_ = idx_copy(1, 1)
        a.wait(); b_.wait()
        gather(1, 1).start()

        def body(i, _):
            slot = lax.rem(i, NS_AGG)
            gather(i, slot).wait()
            scatter(slot).start(add=True)

            @pl.when(i + 2 < NC_AGG)
            def _():
                nslot = lax.rem(i + 2, NS_AGG)
                a, b_ = idx_copy(i + 2, nslot)
                a.wait()
                b_.wait()
                gather(i + 2, nslot).start()

            scatter(slot).wait()

            @pl.when(i + 3 < NC_AGG)
            def _():
                a, b_ = idx_copy(i + 3, slot)
                a.start()
                b_.start()
            return 0

        lax.fori_loop(0, NC_AGG, body, 0)

    run()

    plsc.subcore_barrier()

    @pl.when(c == 0)
    def _():
        pltpu.sync_copy(acc.at[pl.ds(base, ROWS_PER_TILE)],
                        outa.at[pl.ds(base, ROWS_PER_TILE)])

    @pl.when(c == 1)
    def _():
        pltpu.sync_copy(acc.at[pl.ds(base, ROWS_PER_TILE)],
                        outb.at[pl.ds(base, ROWS_PER_TILE)])


@functools.partial(
    pl.kernel,
    out_type=(jax.ShapeDtypeStruct((B, PK), jnp.float32),
              jax.ShapeDtypeStruct((B, PK), jnp.float32)),
    mesh=_MESH,
    compiler_params=_SC_PARAMS,
    scratch_types=[
        pltpu.VMEM((256,), jnp.int32),
        pltpu.VMEM((256, PK), jnp.float32),
        pltpu.SemaphoreType.DMA,
    ],
)
def _sc_decode_gather(x3, uids, iids, outu, outi, idx_v, rows_v, sem):
    c = lax.axis_index("c")
    s = lax.axis_index("s")
    base = s * 256

    @pl.when(c == 0)
    def _():
        pltpu.sync_copy(uids.at[pl.ds(base, 256)], idx_v)

    @pl.when(c == 1)
    def _():
        pltpu.sync_copy(iids.at[pl.ds(base, 256)], idx_v)
        off = jnp.full((16,), N_USERS, jnp.int32)

        def addoff(j, _):
            idx_v[pl.ds(j * 16, 16)] = idx_v[pl.ds(j * 16, 16)] + off
            return 0
        lax.fori_loop(0, 16, addoff, 0)

    pltpu.make_async_copy(x3.at[idx_v], rows_v, sem).start()
    pltpu.make_async_copy(x3.at[idx_v], rows_v, sem).wait()

    @pl.when(c == 0)
    def _():
        pltpu.sync_copy(rows_v, outu.at[pl.ds(base, 256)])

    @pl.when(c == 1)
    def _():
        pltpu.sync_copy(rows_v, outi.at[pl.ds(base, 256)])


_RB = 3128
_GRID = N_PAD // _RB
_PREC = lax.Precision.HIGHEST


def _tc_transform0_body(x_ref, wlT_ref, wrT_ref, bl_ref, t_ref):
    x = x_ref[...]
    y = jnp.dot(x, wlT_ref[...], precision=_PREC)
    z = jnp.dot(x, wrT_ref[...], precision=_PREC) + bl_ref[...]
    t_ref[...] = jnp.concatenate([y, z], axis=1)


def _node_block(acca_ref, accb_ref, dega_ref, degb_ref, tp_ref):
    deg = dega_ref[:, :1] + degb_ref[:, :1]
    inv = 1.0 / jnp.maximum(deg, 1.0)
    agg = jnp.concatenate([acca_ref[...], accb_ref[...]], axis=1) * inv
    return jnp.maximum(agg + tp_ref[:, D:], 0.0)


def _tc_transform_body(acca_ref, accb_ref, dega_ref, degb_ref, tp_ref,
                       wlT_ref, wrT_ref, bl_ref, t_ref):
    x = _node_block(acca_ref, accb_ref, dega_ref, degb_ref, tp_ref)
    y = jnp.dot(x, wlT_ref[...], precision=_PREC)
    z = jnp.dot(x, wrT_ref[...], precision=_PREC) + bl_ref[...]
    t_ref[...] = jnp.concatenate([y, z], axis=1)


def _tc_combine_body(acca_ref, accb_ref, dega_ref, degb_ref, tp_ref, x_ref):
    x = _node_block(acca_ref, accb_ref, dega_ref, degb_ref, tp_ref)
    x_ref[...] = jnp.concatenate([x, x], axis=1)


def _tc_mlp_body(gu_ref, gi_ref, w1uT_ref, w1iT_ref, b1_ref, w2T_ref, b2_ref,
                 out_ref):
    h = (jnp.dot(gu_ref[:, :D], w1uT_ref[...], precision=_PREC)
         + jnp.dot(gi_ref[:, :D], w1iT_ref[...], precision=_PREC)
         + b1_ref[...])
    h = jnp.maximum(h, 0.0)
    r = jnp.dot(h, w2T_ref[...], precision=_PREC) + b2_ref[...]
    out_ref[...] = jnp.clip(r, 1.0, 5.0)


def _row_spec(cols, col0=0):
    cb = col0 // max(cols, 1)
    return pl.BlockSpec((_RB, cols), lambda i, _cb=cb: (i, _cb))


def _full_spec(r, cols):
    return pl.BlockSpec((r, cols), lambda i: (0, 0))


def _tc_transform0(x, wlT, wrT, bl):
    return pl.pallas_call(
        _tc_transform0_body,
        grid=(_GRID,),
        in_specs=[_row_spec(D), _full_spec(D, D), _full_spec(D, D),
                  _full_spec(1, D)],
        out_specs=_row_spec(PK),
        out_shape=jax.ShapeDtypeStruct((N_PAD, PK), jnp.float32),
    )(x, wlT, wrT, bl)


def _tc_transform(acca, accb, dega, degb, tp, wlT, wrT, bl):
    return pl.pallas_call(
        _tc_transform_body,
        grid=(_GRID,),
        in_specs=[_row_spec(DH), _row_spec(DH), _row_spec(16), _row_spec(16),
                  _row_spec(PK), _full_spec(D, D), _full_spec(D, D),
                  _full_spec(1, D)],
        out_specs=_row_spec(PK),
        out_shape=jax.ShapeDtypeStruct((N_PAD, PK), jnp.float32),
    )(acca, accb, dega, degb, tp, wlT, wrT, bl)


def _tc_combine(acca, accb, dega, degb, tp):
    return pl.pallas_call(
        _tc_combine_body,
        grid=(_GRID,),
        in_specs=[_row_spec(DH), _row_spec(DH), _row_spec(16), _row_spec(16),
                  _row_spec(PK)],
        out_specs=_row_spec(PK),
        out_shape=jax.ShapeDtypeStruct((N_PAD, PK), jnp.float32),
    )(acca, accb, dega, degb, tp)


def _tc_mlp(gu, gi, w1uT, w1iT, b1, w2T, b2):
    return pl.pallas_call(
        _tc_mlp_body,
        grid=(1,),
        in_specs=[_full_spec(B, PK), _full_spec(B, PK), _full_spec(D, MLP_H),
                  _full_spec(D, MLP_H), _full_spec(1, MLP_H),
                  _full_spec(MLP_H, 1), _full_spec(1, 1)],
        out_specs=_full_spec(B, 1),
        out_shape=jax.ShapeDtypeStruct((B, 1), jnp.float32),
    )(gu, gi, w1uT, w1iT, b1, w2T, b2)


def kernel(edge_index, user_ids, item_ids, user_emb, item_emb,
           Wl0, bl0, Wr0, Wl1, bl1, Wr1, Wl2, bl2, Wr2,
           W1, b1, W2, b2):
    x0 = jnp.concatenate([user_emb, item_emb], axis=0)
    x0 = jnp.pad(x0, ((0, N_PAD - N_NODES), (0, 0)))
    pad_i = jnp.arange(E_PAD - E, dtype=jnp.int32)
    src_pad = (pad_i * 37) % N_NODES
    dst_pad = N_NODES + pad_i % (N_PAD - N_NODES)
    src_p = jnp.concatenate([edge_index[0], src_pad])
    dst_p = jnp.concatenate([edge_index[1], dst_pad])
    src4 = jnp.stack([src_p * 4, src_p * 4 + 1])

    deg = _sc_degree(dst_p)
    dega, degb = deg[0], deg[1]

    layers = ((Wl0, bl0, Wr0), (Wl1, bl1, Wr1), (Wl2, bl2, Wr2))
    acca = accb = t = None
    for k, (Wl, bl, Wr) in enumerate(layers):
        wlT, wrT, bl2d = Wl.T, Wr.T, bl.reshape(1, D)
        if k == 0:
            t = _tc_transform0(x0, wlT, wrT, bl2d)
        else:
            t = _tc_transform(acca, accb, dega, degb, t, wlT, wrT, bl2d)
        t4 = t.reshape(4 * N_PAD, DH)
        acca, accb = _sc_aggregate(src4, dst_p, t4)

    x3 = _tc_combine(acca, accb, dega, degb, t)
    gu, gi = _sc_decode_gather(x3, user_ids, item_ids)

    w1uT, w1iT = W1[:, :D].T, W1[:, D:].T
    out = _tc_mlp(gu, gi, w1uT, w1iT, b1.reshape(1, MLP_H),
                  W2.T, b2.reshape(1, 1))
    return out.reshape(B)
